# baseline (speedup 1.0000x reference)
import jax
import jax.numpy as jnp
from jax import lax
from jax.experimental import pallas as pl
from jax.experimental.pallas import tpu as pltpu

_HALF_A = 200
_HALF_B = 500


def _pass_a_kernel(a1_ref, a2_ref, x_ref, w1_ref, b1_ref, w10_ref,
                   s2_ref, q_ref, s1_ref):
    g = pl.program_id(0)

    @pl.when(g == 0)
    def _():
        s1_ref[...] = jnp.dot(x_ref[...], w1_ref[...],
                              preferred_element_type=jnp.float32)

    for i, a_ref in enumerate((a1_ref, a2_ref)):
        a = a_ref[0]
        h = jnp.dot(a, s1_ref[...], preferred_element_type=jnp.float32)
        h = jnp.maximum(h + b1_ref[...], 0.0)
        sl = slice(i * _HALF_A, (i + 1) * _HALF_A)
        s2_ref[sl, :] = jnp.dot(h, w10_ref[...],
                                preferred_element_type=jnp.float32)
        q_ref[sl, :] = a.astype(jnp.float8_e4m3fn)


def _pass_b_kernel(q1_ref, q2_ref, s2_ref, b10_ref, out_ref,
                   qs_ref, sc_ref):
    g = pl.program_id(0)

    @pl.when(g == 0)
    def _():
        s2 = s2_ref[...]
        m = jnp.max(jnp.abs(s2), axis=0, keepdims=True)
        sc = jnp.where(m > 0.0, m * (1.0 / 240.0), 1.0)
        qs_ref[...] = (s2 / sc).astype(jnp.float8_e4m3fn)
        sc_ref[...] = sc

    for i, q_ref in enumerate((q1_ref, q2_ref)):
        acc = jnp.dot(q_ref[0], qs_ref[...],
                      preferred_element_type=jnp.float32)
        o = acc * sc_ref[...] + b10_ref[...]
        m = jnp.max(o, axis=1, keepdims=True)
        lse = jnp.log(jnp.sum(jnp.exp(o - m), axis=1, keepdims=True)) + m
        out_ref[slice(i * _HALF_B, (i + 1) * _HALF_B), :] = o - lse


@jax.jit
def kernel(x, adj, W1, b1, W10, b10):
    n, nfeat = x.shape
    nhid = W1.shape[1]
    nclass = W10.shape[1]
    nb = n // (2 * _HALF_A)

    def const(shape):
        return pl.BlockSpec(shape, lambda g: tuple(0 for _ in shape))

    adj3 = adj.reshape(n // _HALF_A, _HALF_A, n)
    half_a = pl.BlockSpec((1, _HALF_A, n), lambda g: (2 * g, 0, 0))
    half_a2 = pl.BlockSpec((1, _HALF_A, n), lambda g: (2 * g + 1, 0, 0))

    s2, q = pl.pallas_call(
        _pass_a_kernel,
        grid=(nb,),
        in_specs=[
            half_a,
            half_a2,
            const((n, nfeat)),
            const((nfeat, nhid)),
            const((1, nhid)),
            const((nhid, nclass)),
        ],
        out_specs=[
            pl.BlockSpec((2 * _HALF_A, nclass), lambda g: (g, 0)),
            pl.BlockSpec((2 * _HALF_A, n), lambda g: (g, 0)),
        ],
        out_shape=[
            jax.ShapeDtypeStruct((n, nclass), jnp.float32),
            jax.ShapeDtypeStruct((n, n), jnp.float8_e4m3fn),
        ],
        scratch_shapes=[pltpu.VMEM((n, nhid), jnp.float32)],
        compiler_params=pltpu.CompilerParams(
            dimension_semantics=("arbitrary",)),
    )(adj3, adj3, x, W1, b1.reshape(1, nhid), W10)

    nb_b = n // (2 * _HALF_B)
    q3 = q.reshape(n // _HALF_B, _HALF_B, n)
    half_b = pl.BlockSpec((1, _HALF_B, n), lambda g: (2 * g, 0, 0))
    half_b2 = pl.BlockSpec((1, _HALF_B, n), lambda g: (2 * g + 1, 0, 0))

    out = pl.pallas_call(
        _pass_b_kernel,
        grid=(nb_b,),
        in_specs=[
            half_b,
            half_b2,
            const((n, nclass)),
            const((1, nclass)),
        ],
        out_specs=pl.BlockSpec((2 * _HALF_B, nclass), lambda g: (g, 0)),
        out_shape=jax.ShapeDtypeStruct((n, nclass), jnp.float32),
        scratch_shapes=[
            pltpu.VMEM((n, nclass), jnp.float8_e4m3fn),
            pltpu.VMEM((1, nclass), jnp.float32),
        ],
        compiler_params=pltpu.CompilerParams(
            dimension_semantics=("arbitrary",)),
    )(q3, q3, s2, b10.reshape(1, nclass))

    return out
